# baseline (speedup 1.0000x reference)
import jax
import jax.numpy as jnp
from jax import lax
from jax.experimental import pallas as pl
from jax.experimental.pallas import tpu as pltpu

_B, _S, _D, _E = 4, 4096, 512, 1024
_H = _E // 2
_K = max(int((_S - 2) * 0.4), 1)
_BS = 512
_NB = _S // _BS
_INTERP = False


def _pcall(*args, **kwargs):
    return pl.pallas_call(*args, interpret=_INTERP, **kwargs)


def _k0(text_ref, maskf_ref, eos_ref, lenc_ref):
    t = text_ref[...]
    mf = (t != 0).astype(jnp.float32)
    maskf_ref[...] = mf
    iota = lax.broadcasted_iota(jnp.int32, (_B, _S), 1)
    mx = jnp.max(t, axis=1, keepdims=True)
    eos = jnp.min(jnp.where(t == mx, iota, _S), axis=1)
    lengths = jnp.sum(mf, axis=1) - 2.0
    lenc = jnp.clip(lengths.astype(jnp.int32), 1, _K)
    for b in range(_B):
        eos_ref[0, b] = eos[b]
        lenc_ref[0, b] = lenc[b]


def _cumsum_lanes(x):
    s = 1
    while s < _S:
        x = x + jnp.pad(x, ((0, 0), (s, 0)))[:, :_S]
        s *= 2
    return x


def _k1(eos_sp, lenc_ref, maskf_ref, att_ref, mk_ref, ml_ref):
    b = pl.program_id(0)
    eosb = eos_sp[0, b]
    r = eosb - (eosb // 8) * 8
    a = att_ref[0, pl.ds(r, 1), :].reshape(1, _S)
    iota = lax.broadcasted_iota(jnp.int32, (1, _S), 1)
    a = jnp.where((iota == eosb) | (iota == 0), jnp.float32(-1.0), a)
    a = a * maskf_ref[0]
    bits = lax.bitcast_convert_type(a, jnp.int32)
    key = jnp.where(bits < 0, bits ^ jnp.int32(0x7FFFFFFF), bits)
    tgt_l = lenc_ref[0, b]
    int_min = jnp.int32(-2147483648)
    int_max = jnp.int32(2147483647)

    def body(_, carry):
        lok, hik, lol, hil = carry
        midk = (lok >> 1) + (hik >> 1) + (lok & hik & 1)
        midl = (lol >> 1) + (hil >> 1) + (lol & hil & 1)
        cntk = jnp.sum((key >= midk).astype(jnp.int32))
        cntl = jnp.sum((key >= midl).astype(jnp.int32))
        gek = cntk >= _K
        gel = cntl >= tgt_l
        lok = jnp.where(gek, midk, lok)
        hik = jnp.where(gek, hik, midk)
        lol = jnp.where(gel, midl, lol)
        hil = jnp.where(gel, hil, midl)
        return lok, hik, lol, hil

    lok, _, lol, _ = lax.fori_loop(
        0, 32, body, (int_min, int_max, int_min, int_max))

    gtk = key > lok
    eqk = key == lok
    ties_k = _K - jnp.sum(gtk.astype(jnp.int32))
    rank_k = _cumsum_lanes(eqk.astype(jnp.int32))
    mk = gtk | (eqk & (rank_k <= ties_k))
    mk_ref[0] = mk.astype(jnp.float32)

    gtl = key > lol
    eql = key == lol
    ties_l = tgt_l - jnp.sum(gtl.astype(jnp.int32))
    rank_l = _cumsum_lanes(eql.astype(jnp.int32))
    ml = gtl | (eql & (rank_l <= ties_l))
    ml_ref[0] = ml.astype(jnp.float32)


def _l2n(f):
    n2 = jnp.sum(f * f, axis=1, keepdims=True)
    return f / (jnp.sqrt(n2) + 1e-8)


def _k2(feat_ref, mk_ref, mlc_ref, w0_ref, b0_ref, w1_ref, b1_ref,
        wlin_ref, blin_ref, g0_ref, beta0_ref, pooled_ref, x_sc, sums_sc):
    p = pl.program_id(0)
    b = pl.program_id(1)
    nb = pl.program_id(2)
    row0 = (b * _NB + nb) * _BS
    sf = _l2n(feat_ref[0]).astype(jnp.bfloat16)

    @pl.when(p == 0)
    def _():
        x = jnp.dot(sf, w0_ref[...], preferred_element_type=jnp.float32)
        x = x + b0_ref[...]
        x_sc[pl.ds(row0, _BS), :] = x

        @pl.when((b == 0) & (nb == 0))
        def _():
            sums_sc[...] = jnp.zeros((2, _H), jnp.float32)

        m = mk_ref[0]
        sums_sc[0:1, :] += jnp.dot(m, x, preferred_element_type=jnp.float32)
        sums_sc[1:2, :] += jnp.dot(m, x * x,
                                   preferred_element_type=jnp.float32)

    @pl.when(p == 1)
    def _():
        inv_n = jnp.float32(1.0 / (_B * _K))
        mu = sums_sc[0:1, :] * inv_n
        ex2 = sums_sc[1:2, :] * inv_n
        var = ex2 - mu * mu
        rstd = lax.rsqrt(var + 1e-5)

        base = jnp.dot(sf, wlin_ref[...], preferred_element_type=jnp.float32)
        base = base + blin_ref[...]

        x = x_sc[pl.ds(row0, _BS), :]
        xn = (x - mu) * (rstd * g0_ref[...]) + beta0_ref[...]
        r = jnp.maximum(xn, 0.0).astype(jnp.bfloat16)
        y = jnp.dot(r, w1_ref[...], preferred_element_type=jnp.float32)
        y = y + b1_ref[...] + base

        mcol = mlc_ref[0]
        ybig = jnp.where(mcol > 0.5, y, -jnp.inf)
        bmax = jnp.max(ybig, axis=0, keepdims=True)

        @pl.when(nb == 0)
        def _():
            pooled_ref[0] = jnp.full((1, _E), -jnp.inf, jnp.float32)

        pooled_ref[0] = jnp.maximum(pooled_ref[0], bmax)


def kernel(features, text, attention, W_lin, b_lin, W0, b0, g0, beta0, W1, b1):
    maskf, eos, lenc = _pcall(
        _k0,
        out_shape=(
            jax.ShapeDtypeStruct((_B, _S), jnp.float32),
            jax.ShapeDtypeStruct((1, _B), jnp.int32),
            jax.ShapeDtypeStruct((1, _B), jnp.int32),
        ),
        in_specs=[pl.BlockSpec(memory_space=pltpu.VMEM)],
        out_specs=(
            pl.BlockSpec(memory_space=pltpu.VMEM),
            pl.BlockSpec(memory_space=pltpu.SMEM),
            pl.BlockSpec(memory_space=pltpu.SMEM),
        ),
    )(text)

    maskf3 = maskf.reshape(_B, 1, _S)

    grid_spec = pltpu.PrefetchScalarGridSpec(
        num_scalar_prefetch=1,
        grid=(_B,),
        in_specs=[
            pl.BlockSpec(memory_space=pltpu.SMEM),
            pl.BlockSpec((1, 1, _S), lambda b, eos_sp: (b, 0, 0)),
            pl.BlockSpec(
                (1, 8, _S),
                lambda b, eos_sp: (b, eos_sp[0, b] // 8, 0)),
        ],
        out_specs=[
            pl.BlockSpec((1, 1, _S), lambda b, eos_sp: (b, 0, 0)),
            pl.BlockSpec((1, 1, _S), lambda b, eos_sp: (b, 0, 0)),
        ],
    )
    mk, ml = _pcall(
        _k1,
        grid_spec=grid_spec,
        out_shape=(
            jax.ShapeDtypeStruct((_B, 1, _S), jnp.float32),
            jax.ShapeDtypeStruct((_B, 1, _S), jnp.float32),
        ),
    )(eos, lenc, maskf3, attention)

    ml_col = ml.reshape(_B, _S, 1)

    w0h = W0.astype(jnp.bfloat16)
    w1h = W1.astype(jnp.bfloat16)
    wlinh = W_lin.astype(jnp.bfloat16)
    b0r = b0.reshape(1, _H)
    g0r = g0.reshape(1, _H)
    beta0r = beta0.reshape(1, _H)
    b1r = b1.reshape(1, _E)
    blinr = b_lin.reshape(1, _E)

    pooled = _pcall(
        _k2,
        grid=(2, _B, _NB),
        in_specs=[
            pl.BlockSpec((1, _BS, _D), lambda p, b, nb: (b, nb, 0)),
            pl.BlockSpec((1, 1, _BS), lambda p, b, nb: (b, 0, nb)),
            pl.BlockSpec((1, _BS, 1), lambda p, b, nb: (b, nb, 0)),
            pl.BlockSpec((_D, _H), lambda p, b, nb: (0, 0)),
            pl.BlockSpec((1, _H), lambda p, b, nb: (0, 0)),
            pl.BlockSpec((_H, _E), lambda p, b, nb: (0, 0)),
            pl.BlockSpec((1, _E), lambda p, b, nb: (0, 0)),
            pl.BlockSpec((_D, _E), lambda p, b, nb: (0, 0)),
            pl.BlockSpec((1, _E), lambda p, b, nb: (0, 0)),
            pl.BlockSpec((1, _H), lambda p, b, nb: (0, 0)),
            pl.BlockSpec((1, _H), lambda p, b, nb: (0, 0)),
        ],
        out_specs=pl.BlockSpec((1, 1, _E), lambda p, b, nb: (b, 0, 0)),
        out_shape=jax.ShapeDtypeStruct((_B, 1, _E), jnp.float32),
        scratch_shapes=[
            pltpu.VMEM((_B * _S, _H), jnp.float32),
            pltpu.VMEM((2, _H), jnp.float32),
        ],
    )(features, mk, ml_col, w0h, b0r, w1h, b1r, wlinh, blinr, g0r, beta0r)

    return pooled.reshape(_B, _E)


# speedup vs baseline: 1.0286x; 1.0286x over previous
import jax
import jax.numpy as jnp
from jax import lax
from jax.experimental import pallas as pl
from jax.experimental.pallas import tpu as pltpu

_B, _S, _D, _E = 4, 4096, 512, 1024
_H = _E // 2
_K = max(int((_S - 2) * 0.4), 1)
_BS = 512
_NB = _S // _BS
_INTERP = False


def _pcall(*args, **kwargs):
    return pl.pallas_call(*args, interpret=_INTERP, **kwargs)


def _k0(text_ref, maskf_ref, eos_ref, lenc_ref):
    t = text_ref[...]
    mf = (t != 0).astype(jnp.float32)
    maskf_ref[...] = mf
    iota = lax.broadcasted_iota(jnp.int32, (_B, _S), 1)
    mx = jnp.max(t, axis=1, keepdims=True)
    eos = jnp.min(jnp.where(t == mx, iota, _S), axis=1)
    lengths = jnp.sum(mf, axis=1) - 2.0
    lenc = jnp.clip(lengths.astype(jnp.int32), 1, _K)
    for b in range(_B):
        eos_ref[0, b] = eos[b]
        lenc_ref[0, b] = lenc[b]


def _cumsum_lanes(x):
    s = 1
    while s < _S:
        x = x + jnp.pad(x, ((0, 0), (s, 0)))[:, :_S]
        s *= 2
    return x


def _k1(eos_sp, lenc_ref, maskf_ref, att_ref, mk_ref, ml_ref):
    b = pl.program_id(0)
    eosb = eos_sp[0, b]
    r = eosb - (eosb // 8) * 8
    a = att_ref[0, pl.ds(r, 1), :].reshape(1, _S)
    iota = lax.broadcasted_iota(jnp.int32, (1, _S), 1)
    a = jnp.where((iota == eosb) | (iota == 0), jnp.float32(-1.0), a)
    a = a * maskf_ref[0]
    bits = lax.bitcast_convert_type(a, jnp.int32)
    key = jnp.where(bits < 0, bits ^ jnp.int32(0x7FFFFFFF), bits)
    tgt_l = lenc_ref[0, b]
    int_min = jnp.int32(-2147483648)
    int_max = jnp.int32(2147483647)

    def body(_, carry):
        lok, hik, lol, hil = carry
        midk = (lok >> 1) + (hik >> 1) + (lok & hik & 1)
        midl = (lol >> 1) + (hil >> 1) + (lol & hil & 1)
        cntk = jnp.sum((key >= midk).astype(jnp.int32))
        cntl = jnp.sum((key >= midl).astype(jnp.int32))
        gek = cntk >= _K
        gel = cntl >= tgt_l
        lok = jnp.where(gek, midk, lok)
        hik = jnp.where(gek, hik, midk)
        lol = jnp.where(gel, midl, lol)
        hil = jnp.where(gel, hil, midl)
        return lok, hik, lol, hil

    lok, _, lol, _ = lax.fori_loop(
        0, 32, body, (int_min, int_max, int_min, int_max))

    gtk = key > lok
    eqk = key == lok
    ties_k = _K - jnp.sum(gtk.astype(jnp.int32))
    rank_k = _cumsum_lanes(eqk.astype(jnp.int32))
    mk = gtk | (eqk & (rank_k <= ties_k))
    mk_ref[0] = mk.astype(jnp.float32)

    gtl = key > lol
    eql = key == lol
    ties_l = tgt_l - jnp.sum(gtl.astype(jnp.int32))
    rank_l = _cumsum_lanes(eql.astype(jnp.int32))
    ml = gtl | (eql & (rank_l <= ties_l))
    ml_ref[0] = ml.astype(jnp.float32)


def _l2n(f):
    n2 = jnp.sum(f * f, axis=1, keepdims=True)
    return f / (jnp.sqrt(n2) + 1e-8)


def _k2(feat_ref, mk_ref, mlc_ref, w0_ref, b0_ref, w1_ref, b1_ref,
        wlin_ref, blin_ref, g0_ref, beta0_ref, pooled_ref, x_sc, sums_sc):
    p = pl.program_id(0)
    b = pl.program_id(1)
    nb = pl.program_id(2)
    row0 = (b * _NB + nb) * _BS
    sf = _l2n(feat_ref[0])

    @pl.when(p == 0)
    def _():
        x = jnp.dot(sf, w0_ref[...], preferred_element_type=jnp.float32)
        x = x + b0_ref[...]
        x_sc[pl.ds(row0, _BS), :] = x

        @pl.when((b == 0) & (nb == 0))
        def _():
            sums_sc[...] = jnp.zeros((2, _H), jnp.float32)

        m = mk_ref[0]
        sums_sc[0:1, :] += jnp.dot(m, x, preferred_element_type=jnp.float32)
        sums_sc[1:2, :] += jnp.dot(m, x * x,
                                   preferred_element_type=jnp.float32)

    @pl.when(p == 1)
    def _():
        inv_n = jnp.float32(1.0 / (_B * _K))
        mu = sums_sc[0:1, :] * inv_n
        ex2 = sums_sc[1:2, :] * inv_n
        var = ex2 - mu * mu
        rstd = lax.rsqrt(var + 1e-5)

        base = jnp.dot(sf, wlin_ref[...], preferred_element_type=jnp.float32)
        base = base + blin_ref[...]

        x = x_sc[pl.ds(row0, _BS), :]
        xn = (x - mu) * (rstd * g0_ref[...]) + beta0_ref[...]
        r = jnp.maximum(xn, 0.0)
        y = jnp.dot(r, w1_ref[...], preferred_element_type=jnp.float32)
        y = y + b1_ref[...] + base

        mcol = mlc_ref[0]
        ybig = jnp.where(mcol > 0.5, y, -jnp.inf)
        bmax = jnp.max(ybig, axis=0, keepdims=True)

        @pl.when(nb == 0)
        def _():
            pooled_ref[0] = jnp.full((1, _E), -jnp.inf, jnp.float32)

        pooled_ref[0] = jnp.maximum(pooled_ref[0], bmax)


def kernel(features, text, attention, W_lin, b_lin, W0, b0, g0, beta0, W1, b1):
    maskf, eos, lenc = _pcall(
        _k0,
        out_shape=(
            jax.ShapeDtypeStruct((_B, _S), jnp.float32),
            jax.ShapeDtypeStruct((1, _B), jnp.int32),
            jax.ShapeDtypeStruct((1, _B), jnp.int32),
        ),
        in_specs=[pl.BlockSpec(memory_space=pltpu.VMEM)],
        out_specs=(
            pl.BlockSpec(memory_space=pltpu.VMEM),
            pl.BlockSpec(memory_space=pltpu.SMEM),
            pl.BlockSpec(memory_space=pltpu.SMEM),
        ),
    )(text)

    maskf3 = maskf.reshape(_B, 1, _S)

    grid_spec = pltpu.PrefetchScalarGridSpec(
        num_scalar_prefetch=1,
        grid=(_B,),
        in_specs=[
            pl.BlockSpec(memory_space=pltpu.SMEM),
            pl.BlockSpec((1, 1, _S), lambda b, eos_sp: (b, 0, 0)),
            pl.BlockSpec(
                (1, 8, _S),
                lambda b, eos_sp: (b, eos_sp[0, b] // 8, 0)),
        ],
        out_specs=[
            pl.BlockSpec((1, 1, _S), lambda b, eos_sp: (b, 0, 0)),
            pl.BlockSpec((1, 1, _S), lambda b, eos_sp: (b, 0, 0)),
        ],
    )
    mk, ml = _pcall(
        _k1,
        grid_spec=grid_spec,
        out_shape=(
            jax.ShapeDtypeStruct((_B, 1, _S), jnp.float32),
            jax.ShapeDtypeStruct((_B, 1, _S), jnp.float32),
        ),
    )(eos, lenc, maskf3, attention)

    ml_col = ml.reshape(_B, _S, 1)

    b0r = b0.reshape(1, _H)
    g0r = g0.reshape(1, _H)
    beta0r = beta0.reshape(1, _H)
    b1r = b1.reshape(1, _E)
    blinr = b_lin.reshape(1, _E)

    pooled = _pcall(
        _k2,
        grid=(2, _B, _NB),
        in_specs=[
            pl.BlockSpec((1, _BS, _D), lambda p, b, nb: (b, nb, 0)),
            pl.BlockSpec((1, 1, _BS), lambda p, b, nb: (b, 0, nb)),
            pl.BlockSpec((1, _BS, 1), lambda p, b, nb: (b, nb, 0)),
            pl.BlockSpec((_D, _H), lambda p, b, nb: (0, 0)),
            pl.BlockSpec((1, _H), lambda p, b, nb: (0, 0)),
            pl.BlockSpec((_H, _E), lambda p, b, nb: (0, 0)),
            pl.BlockSpec((1, _E), lambda p, b, nb: (0, 0)),
            pl.BlockSpec((_D, _E), lambda p, b, nb: (0, 0)),
            pl.BlockSpec((1, _E), lambda p, b, nb: (0, 0)),
            pl.BlockSpec((1, _H), lambda p, b, nb: (0, 0)),
            pl.BlockSpec((1, _H), lambda p, b, nb: (0, 0)),
        ],
        out_specs=pl.BlockSpec((1, 1, _E), lambda p, b, nb: (b, 0, 0)),
        out_shape=jax.ShapeDtypeStruct((_B, 1, _E), jnp.float32),
        scratch_shapes=[
            pltpu.VMEM((_B * _S, _H), jnp.float32),
            pltpu.VMEM((2, _H), jnp.float32),
        ],
    )(features, mk, ml_col, W0, b0r, W1, b1r, W_lin, blinr, g0r, beta0r)

    return pooled.reshape(_B, _E)
